# Initial kernel scaffold; baseline (speedup 1.0000x reference)
import jax
import jax.numpy as jnp
from jax.experimental import pallas as pl


def kernel(x, edge_index, etypes, node_graph_ids, lin_W, lin_b, gru_w_ih, gru_w_hh, gru_b_ih, gru_b_hh, out_W, out_b):
    raise NotImplementedError("write your pallas kernel here")


# speedup vs baseline: 5.0221x; 5.0221x over previous
import functools

import jax
import jax.numpy as jnp
from jax import lax
from jax.experimental import pallas as pl
from jax.experimental.pallas import tpu as pltpu
from jax.experimental.pallas import tpu_sc as plsc

N = 10000
D = 256
NL = 2
NE = 4
NS = 3
G = 64
C = 10

HALF = 128
NPAD = 10240
EPAD = 163840
CHUNK = 128
NTILES = 16
NCHUNK = EPAD // NTILES // CHUNK
KB = 16
NKB = NCHUNK // KB
ROWS_PER_TILE = NPAD // NTILES
TBL = 2 * NE * NPAD

BN = 512
NPO = NPAD // BN


def _trans_body(h_ref, w_ref, b_ref, out_ref):
    acc = lax.dot_general(h_ref[...], w_ref[0, 0],
                          (((1,), (1,)), ((), ())),
                          preferred_element_type=jnp.float32)
    out_ref[...] = acc + b_ref[0, 0]


def _trans_table(h, W, b):
    grid = (NPO, NE, 2)
    return pl.pallas_call(
        _trans_body,
        grid=grid,
        in_specs=[
            pl.BlockSpec((BN, D), lambda i, t, hh: (i, 0)),
            pl.BlockSpec((1, 1, HALF, D), lambda i, t, hh: (t, hh, 0, 0)),
            pl.BlockSpec((1, 1, 1, HALF), lambda i, t, hh: (t, hh, 0, 0)),
        ],
        out_specs=pl.BlockSpec(
            (BN, HALF), lambda i, t, hh: (hh * (NE * NPO) + t * NPO + i, 0)),
        out_shape=jax.ShapeDtypeStruct((TBL, HALF), jnp.float32),
    )(h, W.reshape(NE, 2, HALF, D), b.reshape(NE, 2, 1, HALF))


def _sc_segment_sum(table, gidx2, dst2, zeros_block):
    mesh = plsc.VectorSubcoreMesh(core_axis_name="c", subcore_axis_name="s")

    @functools.partial(
        pl.kernel,
        mesh=mesh,
        out_type=jax.ShapeDtypeStruct((2, NPAD, HALF), jnp.float32),
        scratch_types=[
            pltpu.VMEM((KB, CHUNK), jnp.int32),
            pltpu.VMEM((KB, CHUNK), jnp.int32),
            pltpu.VMEM((CHUNK, HALF), jnp.float32),
            pltpu.VMEM((CHUNK, HALF), jnp.float32),
            pltpu.VMEM_SHARED((NPAD, HALF), jnp.float32),
            pltpu.SemaphoreType.DMA,
            pltpu.SemaphoreType.DMA,
        ],
    )
    def seg_kernel(table_hbm, gidx_hbm, dst_hbm, zero_hbm, out_hbm,
                   gi_v, di_v, buf0, buf1, acc_sh, sem0, sem1):
        c = lax.axis_index("c")
        s = lax.axis_index("s")
        pltpu.sync_copy(zero_hbm, acc_sh.at[pl.ds(s * ROWS_PER_TILE,
                                                  ROWS_PER_TILE)])
        plsc.subcore_barrier()

        @pl.loop(0, NKB)
        def _(kb):
            pltpu.sync_copy(gidx_hbm.at[c, s, kb], gi_v)
            pltpu.sync_copy(dst_hbm.at[s, kb], di_v)
            pltpu.async_copy(table_hbm.at[gi_v.at[0]], buf0, sem0)

            @pl.loop(0, KB, step=2)
            def _(j):
                pltpu.make_async_copy(table_hbm.at[gi_v.at[j]], buf0,
                                      sem0).wait()
                pltpu.async_copy(table_hbm.at[gi_v.at[j + 1]], buf1, sem1)
                pltpu.sync_copy(buf0, acc_sh.at[di_v.at[j]], add=True)
                pltpu.make_async_copy(table_hbm.at[gi_v.at[j + 1]], buf1,
                                      sem1).wait()

                @pl.when(j + 2 < KB)
                def _():
                    pltpu.async_copy(table_hbm.at[gi_v.at[j + 2]], buf0, sem0)

                pltpu.sync_copy(buf1, acc_sh.at[di_v.at[j + 1]], add=True)

        plsc.subcore_barrier()
        pltpu.sync_copy(
            acc_sh.at[pl.ds(s * ROWS_PER_TILE, ROWS_PER_TILE)],
            out_hbm.at[c].at[pl.ds(s * ROWS_PER_TILE, ROWS_PER_TILE)])

    return seg_kernel(table, gidx2, dst2, zeros_block)


def _gru_body(a_ref, h_ref, wih_ref, whh_ref, bih_ref, bhh_ref, out_ref):
    a = jnp.concatenate([a_ref[0], a_ref[1]], axis=1)
    h = h_ref[...]
    gi = lax.dot_general(a, wih_ref[...], (((1,), (1,)), ((), ())),
                         preferred_element_type=jnp.float32) + bih_ref[...]
    gh = lax.dot_general(h, whh_ref[...], (((1,), (1,)), ((), ())),
                         preferred_element_type=jnp.float32) + bhh_ref[...]
    r = jax.nn.sigmoid(gi[:, :D] + gh[:, :D])
    z = jax.nn.sigmoid(gi[:, D:2 * D] + gh[:, D:2 * D])
    n = jnp.tanh(gi[:, 2 * D:] + r * gh[:, 2 * D:])
    out_ref[...] = (1.0 - z) * n + z * h


def _gru(a2, h, w_ih, w_hh, b_ih, b_hh):
    return pl.pallas_call(
        _gru_body,
        grid=(NPO,),
        in_specs=[
            pl.BlockSpec((2, BN, HALF), lambda i: (0, i, 0)),
            pl.BlockSpec((BN, D), lambda i: (i, 0)),
            pl.BlockSpec((3 * D, D), lambda i: (0, 0)),
            pl.BlockSpec((3 * D, D), lambda i: (0, 0)),
            pl.BlockSpec((1, 3 * D), lambda i: (0, 0)),
            pl.BlockSpec((1, 3 * D), lambda i: (0, 0)),
        ],
        out_specs=pl.BlockSpec((BN, D), lambda i: (i, 0)),
        out_shape=jax.ShapeDtypeStruct((NPAD, D), jnp.float32),
    )(a2, h, w_ih, w_hh, b_ih.reshape(1, 3 * D), b_hh.reshape(1, 3 * D))


def _pool_body(h_ref, ids_ref, wt_ref, b_ref, out_ref):
    ids = ids_ref[...]
    ohT = (lax.broadcasted_iota(jnp.int32, (G, N), 0)
           == jnp.broadcast_to(ids, (G, N))).astype(jnp.float32)
    pooled = lax.dot_general(ohT, h_ref[...], (((1,), (0,)), ((), ())),
                             preferred_element_type=jnp.float32)
    counts = jnp.sum(ohT, axis=1, keepdims=True)
    hg = pooled / jnp.maximum(counts, 1.0)
    out_ref[...] = lax.dot_general(hg, wt_ref[...], (((1,), (0,)), ((), ())),
                                   preferred_element_type=jnp.float32) \
        + b_ref[...]


def _pool_logits(h, ids, out_W, out_b):
    wt_pad = jnp.zeros((D, 128), jnp.float32).at[:, :C].set(out_W.T)
    b_pad = jnp.zeros((1, 128), jnp.float32).at[0, :C].set(out_b)
    logits_pad = pl.pallas_call(
        _pool_body,
        in_specs=[
            pl.BlockSpec((N, D), lambda: (0, 0)),
            pl.BlockSpec((1, N), lambda: (0, 0)),
            pl.BlockSpec((D, 128), lambda: (0, 0)),
            pl.BlockSpec((1, 128), lambda: (0, 0)),
        ],
        out_specs=pl.BlockSpec((G, 128), lambda: (0, 0)),
        out_shape=jax.ShapeDtypeStruct((G, 128), jnp.float32),
    )(h, ids.reshape(1, N), wt_pad, b_pad)
    return logits_pad[:, :C]


def kernel(x, edge_index, etypes, node_graph_ids, lin_W, lin_b,
           gru_w_ih, gru_w_hh, gru_b_ih, gru_b_hh, out_W, out_b):
    src = edge_index[0]
    dst = edge_index[1]

    W = lin_W[NL - 1]
    b = lin_b[NL - 1]
    w_ih = gru_w_ih[NL - 1]
    w_hh = gru_w_hh[NL - 1]
    b_ih = gru_b_ih[NL - 1]
    b_hh = gru_b_hh[NL - 1]

    gidx = etypes.astype(jnp.int32) * NPAD + src
    pad_e = EPAD - gidx.shape[0]
    gidx_p = jnp.concatenate([gidx, jnp.zeros((pad_e,), jnp.int32)])
    dst_p = jnp.concatenate(
        [dst, jnp.full((pad_e,), NPAD - 1, jnp.int32)])
    gidx2 = jnp.stack([gidx_p, gidx_p + NE * NPAD]).reshape(
        2, NTILES, NKB, KB, CHUNK)
    dst2 = dst_p.reshape(NTILES, NKB, KB, CHUNK)
    zeros_block = jnp.zeros((ROWS_PER_TILE, HALF), jnp.float32)

    h = jnp.zeros((NPAD, D), jnp.float32).at[:N].set(x.astype(jnp.float32))
    for _ in range(NS):
        table = _trans_table(h, W, b)
        a2 = _sc_segment_sum(table, gidx2, dst2, zeros_block)
        h = _gru(a2, h, w_ih, w_hh, b_ih, b_hh)

    return _pool_logits(h[:N], node_graph_ids.astype(jnp.int32), out_W, out_b)
